# Initial kernel scaffold; baseline (speedup 1.0000x reference)
import jax
import jax.numpy as jnp
from jax.experimental import pallas as pl


def kernel(det_boxes, det_scores, det_labels, det_image_ids, gt_boxes, gt_labels, gt_image_ids):
    raise NotImplementedError("write your pallas kernel here")


# speedup vs baseline: 185.4397x; 185.4397x over previous
import jax
import jax.numpy as jnp
from jax.experimental import pallas as pl
from jax.experimental.pallas import tpu as pltpu

NUM_CLASSES = 80
CPC = NUM_CLASSES // 2
IOU_THRESHOLD = 0.5
EPS = 1e-6


def _map_kernel(starts_ref, counts_ref, det_ref, dl_ref, gt_ref, out_ref, gat_ref):
    core = pl.program_id(0)
    cbase = core * CPC

    g = gt_ref[...]
    gx1 = g[0:1, :]
    gy1 = g[1:2, :]
    gx2 = g[2:3, :]
    gy2 = g[3:4, :]
    glab = g[4:5, :]
    gimg = g[5:6, :]
    garea = jnp.abs((gx2 - gx1) * (gy2 - gy1))
    G = g.shape[1]

    class_vec = cbase + jax.lax.broadcasted_iota(jnp.int32, (CPC, 1), 0)
    class_f = class_vec.astype(jnp.float32)
    class_mask = glab == class_f
    n_gt = jnp.sum(jnp.where(class_mask, 1.0, 0.0), axis=1, keepdims=True)

    dl = dl_ref[...]
    counts_vec = jnp.sum(
        jnp.where(dl == class_vec, 1.0, 0.0), axis=1, keepdims=True
    )

    lane_g = jax.lax.broadcasted_iota(jnp.int32, (1, G), 1)
    ndet = det_ref.shape[0]

    maxc = jax.lax.fori_loop(
        0, CPC, lambda i, m: jnp.maximum(m, counts_ref[cbase + i]), 0
    )

    def body(r, carry):
        matched, tpc, fpc, rp, pp, ap = carry

        for ci in range(CPC):
            pos = jnp.minimum(starts_ref[cbase + ci] + r, ndet - 1)
            gat_ref[ci : ci + 1, :] = det_ref[pl.ds(pos, 1), :]
        gat = gat_ref[...]
        dx1 = gat[:, 0:1]
        dy1 = gat[:, 1:2]
        dx2 = gat[:, 2:3]
        dy2 = gat[:, 3:4]
        dimg = gat[:, 4:5]
        darea = jnp.abs((dx2 - dx1) * (dy2 - dy1))

        ix1 = jnp.maximum(dx1, gx1)
        iy1 = jnp.maximum(dy1, gy1)
        ix2 = jnp.minimum(dx2, gx2)
        iy2 = jnp.minimum(dy2, gy2)
        inter = jnp.clip(ix2 - ix1, 0.0) * jnp.clip(iy2 - iy1, 0.0)
        iou = inter / (darea + garea - inter + EPS)

        active = counts_vec > r
        valid = class_mask & (gimg == dimg) & active
        iou_m = jnp.where(valid, iou, -1.0)

        best = jnp.argmax(iou_m, axis=1, keepdims=True)
        bval = jnp.max(iou_m, axis=1, keepdims=True)
        B = lane_g == best
        mb = jnp.max(jnp.where(B, matched, 0.0), axis=1, keepdims=True)
        is_tp = (bval > IOU_THRESHOLD) & (mb < 0.5)
        tp_f = jnp.where(is_tp, 1.0, 0.0)
        matched = jnp.maximum(
            matched,
            jnp.max(jnp.where(B & is_tp, 1.0, 0.0), axis=0, keepdims=True),
        )

        tpc = tpc + tp_f
        fpc = fpc + jnp.where(active, 1.0, 0.0) - tp_f
        recall = tpc / (n_gt + EPS)
        denom = tpc + fpc
        prec = jnp.where(denom > 0, tpc / (denom + EPS), 1.0)
        ap = ap + (recall - rp) * (prec + pp) * 0.5
        return matched, tpc, fpc, recall, prec, ap

    zc = jnp.zeros((CPC, 1), jnp.float32)
    init = (
        jnp.zeros((1, G), jnp.float32),
        zc,
        zc,
        zc,
        jnp.ones((CPC, 1), jnp.float32),
        zc,
    )
    carry = jax.lax.fori_loop(0, maxc, body, init)
    ap = carry[5]

    has_gt = n_gt > 0
    ap_sum = jnp.sum(jnp.where(has_gt, ap, 0.0))
    hg_cnt = jnp.sum(jnp.where(has_gt, 1.0, 0.0))
    lane = jax.lax.broadcasted_iota(jnp.int32, (1, 8, 128), 2)
    out_ref[...] = jnp.where(lane == 0, ap_sum, jnp.where(lane == 1, hg_cnt, 0.0))


def kernel(det_boxes, det_scores, det_labels, det_image_ids, gt_boxes, gt_labels, gt_image_ids):
    D = det_boxes.shape[0]
    G = gt_boxes.shape[0]

    order = jnp.lexsort((-det_scores, det_labels))
    db = det_boxes[order]
    dl = det_labels[order].astype(jnp.int32)
    di = det_image_ids[order]

    det_pack = jnp.concatenate(
        [db, di.astype(jnp.float32)[:, None], jnp.zeros((D, 3), jnp.float32)], axis=1
    )
    gt_pack = jnp.concatenate(
        [
            gt_boxes.T,
            gt_labels.astype(jnp.float32)[None, :],
            gt_image_ids.astype(jnp.float32)[None, :],
            jnp.zeros((2, G), jnp.float32),
        ],
        axis=0,
    )

    classes = jnp.arange(NUM_CLASSES, dtype=jnp.int32)
    starts = jnp.searchsorted(dl, classes).astype(jnp.int32)
    ends = jnp.searchsorted(dl, classes, side="right").astype(jnp.int32)
    counts = ends - starts

    out = pl.pallas_call(
        _map_kernel,
        grid=(2,),
        in_specs=[
            pl.BlockSpec(memory_space=pltpu.SMEM),
            pl.BlockSpec(memory_space=pltpu.SMEM),
            pl.BlockSpec((D, 8), lambda i: (0, 0)),
            pl.BlockSpec((1, D), lambda i: (0, 0)),
            pl.BlockSpec((8, G), lambda i: (0, 0)),
        ],
        out_specs=pl.BlockSpec((1, 8, 128), lambda i: (i, 0, 0)),
        out_shape=jax.ShapeDtypeStruct((2, 8, 128), jnp.float32),
        scratch_shapes=[pltpu.VMEM((CPC, 8), jnp.float32)],
        compiler_params=pltpu.CompilerParams(
            dimension_semantics=("parallel",),
        ),
    )(starts, counts, det_pack, dl[None, :], gt_pack)

    ap_total = out[0, 0, 0] + out[1, 0, 0]
    cnt_total = out[0, 0, 1] + out[1, 0, 1]
    return ap_total / jnp.maximum(cnt_total, 1.0)


# speedup vs baseline: 194.7058x; 1.0500x over previous
import jax
import jax.numpy as jnp
from jax.experimental import pallas as pl
from jax.experimental.pallas import tpu as pltpu

NUM_CLASSES = 80
CPC = NUM_CLASSES // 2
IOU_THRESHOLD = 0.5
EPS = 1e-6
DB = 64


def _map_kernel(starts_ref, counts_ref, det_ref, dl_ref, gt_ref, out_ref, vb_ref, gat_ref):
    core = pl.program_id(0)
    cbase = core * CPC

    g = gt_ref[...]
    gx1 = g[0:1, :]
    gy1 = g[1:2, :]
    gx2 = g[2:3, :]
    gy2 = g[3:4, :]
    glab = g[4:5, :]
    gimg = g[5:6, :]
    garea = jnp.abs((gx2 - gx1) * (gy2 - gy1))
    G = g.shape[1]
    D = det_ref.shape[0]

    class_vec = cbase + jax.lax.broadcasted_iota(jnp.int32, (CPC, 1), 0)
    class_f = class_vec.astype(jnp.float32)
    class_mask = glab == class_f
    n_gt = jnp.sum(jnp.where(class_mask, 1.0, 0.0), axis=1, keepdims=True)

    dl = dl_ref[...]
    counts_vec = jnp.sum(
        jnp.where(dl == class_vec, 1.0, 0.0), axis=1, keepdims=True
    ).astype(jnp.float32)

    lane_g = jax.lax.broadcasted_iota(jnp.int32, (1, G), 1)

    s_lo = starts_ref[cbase]
    s_hi = jnp.where(cbase + CPC >= NUM_CLASSES, D,
                     starts_ref[jnp.minimum(cbase + CPC, NUM_CLASSES - 1)])

    def blockA(b, _):
        row0 = b * DB
        d = det_ref[pl.ds(row0, DB), :]
        dx1 = d[:, 0:1]
        dy1 = d[:, 1:2]
        dx2 = d[:, 2:3]
        dy2 = d[:, 3:4]
        dimg = d[:, 4:5]
        dlab = d[:, 5:6]
        darea = jnp.abs((dx2 - dx1) * (dy2 - dy1))

        ix1 = jnp.maximum(dx1, gx1)
        iy1 = jnp.maximum(dy1, gy1)
        ix2 = jnp.minimum(dx2, gx2)
        iy2 = jnp.minimum(dy2, gy2)
        inter = jnp.clip(ix2 - ix1, 0.0) * jnp.clip(iy2 - iy1, 0.0)
        iou = inter / (darea + garea - inter + EPS)
        valid = (glab == dlab) & (gimg == dimg)
        iou_m = jnp.where(valid, iou, -1.0)
        val = jnp.max(iou_m, axis=1, keepdims=True)
        best = jnp.argmax(iou_m, axis=1, keepdims=True)
        vb_ref[pl.ds(row0, DB), 0:1] = val
        vb_ref[pl.ds(row0, DB), 1:2] = best.astype(jnp.float32)
        return 0

    jax.lax.fori_loop(s_lo // DB, (s_hi + DB - 1) // DB, blockA, 0)

    maxc = jax.lax.fori_loop(
        0, CPC, lambda i, m: jnp.maximum(m, counts_ref[cbase + i]), 0
    )

    def body(r, carry):
        matched, tpc, fpc, rp, pp, ap = carry

        for ci in range(CPC):
            pos = jnp.minimum(starts_ref[cbase + ci] + r, D - 1)
            gat_ref[ci : ci + 1, :] = vb_ref[pl.ds(pos, 1), :]
        gat = gat_ref[...]
        active = counts_vec > r
        bval = jnp.where(active, gat[:, 0:1], -1.0)
        best = gat[:, 1:2].astype(jnp.int32)

        B = lane_g == best
        mb = jnp.max(jnp.where(B, matched, 0.0), axis=1, keepdims=True)
        is_tp = (bval > IOU_THRESHOLD) & (mb < 0.5)
        tp_f = jnp.where(is_tp, 1.0, 0.0)
        matched = jnp.maximum(
            matched,
            jnp.max(jnp.where(B & is_tp, 1.0, 0.0), axis=0, keepdims=True),
        )

        tpc = tpc + tp_f
        fpc = fpc + jnp.where(active, 1.0, 0.0) - tp_f
        recall = tpc / (n_gt + EPS)
        denom = tpc + fpc
        prec = jnp.where(denom > 0, tpc / (denom + EPS), 1.0)
        ap = ap + (recall - rp) * (prec + pp) * 0.5
        return matched, tpc, fpc, recall, prec, ap

    zc = jnp.zeros((CPC, 1), jnp.float32)
    init = (
        jnp.zeros((1, G), jnp.float32),
        zc,
        zc,
        zc,
        jnp.ones((CPC, 1), jnp.float32),
        zc,
    )
    carry = jax.lax.fori_loop(0, maxc, body, init)
    ap = carry[5]

    has_gt = n_gt > 0
    ap_sum = jnp.sum(jnp.where(has_gt, ap, 0.0))
    hg_cnt = jnp.sum(jnp.where(has_gt, 1.0, 0.0))
    lane = jax.lax.broadcasted_iota(jnp.int32, (1, 8, 128), 2)
    out_ref[...] = jnp.where(lane == 0, ap_sum, jnp.where(lane == 1, hg_cnt, 0.0))


def kernel(det_boxes, det_scores, det_labels, det_image_ids, gt_boxes, gt_labels, gt_image_ids):
    D = det_boxes.shape[0]
    G = gt_boxes.shape[0]

    order = jnp.lexsort((-det_scores, det_labels))
    db = det_boxes[order]
    dl = det_labels[order].astype(jnp.int32)
    di = det_image_ids[order]

    det_pack = jnp.concatenate(
        [
            db,
            di.astype(jnp.float32)[:, None],
            dl.astype(jnp.float32)[:, None],
            jnp.zeros((D, 2), jnp.float32),
        ],
        axis=1,
    )
    gt_pack = jnp.concatenate(
        [
            gt_boxes.T,
            gt_labels.astype(jnp.float32)[None, :],
            gt_image_ids.astype(jnp.float32)[None, :],
            jnp.zeros((2, G), jnp.float32),
        ],
        axis=0,
    )

    classes = jnp.arange(NUM_CLASSES, dtype=jnp.int32)
    starts = jnp.searchsorted(dl, classes).astype(jnp.int32)
    ends = jnp.searchsorted(dl, classes, side="right").astype(jnp.int32)
    counts = ends - starts

    out = pl.pallas_call(
        _map_kernel,
        grid=(2,),
        in_specs=[
            pl.BlockSpec(memory_space=pltpu.SMEM),
            pl.BlockSpec(memory_space=pltpu.SMEM),
            pl.BlockSpec((D, 8), lambda i: (0, 0)),
            pl.BlockSpec((1, D), lambda i: (0, 0)),
            pl.BlockSpec((8, G), lambda i: (0, 0)),
        ],
        out_specs=pl.BlockSpec((1, 8, 128), lambda i: (i, 0, 0)),
        out_shape=jax.ShapeDtypeStruct((2, 8, 128), jnp.float32),
        scratch_shapes=[
            pltpu.VMEM((D, 8), jnp.float32),
            pltpu.VMEM((CPC, 8), jnp.float32),
        ],
        compiler_params=pltpu.CompilerParams(
            dimension_semantics=("parallel",),
        ),
    )(starts, counts, det_pack, dl[None, :], gt_pack)

    ap_total = out[0, 0, 0] + out[1, 0, 0]
    cnt_total = out[0, 0, 1] + out[1, 0, 1]
    return ap_total / jnp.maximum(cnt_total, 1.0)


# speedup vs baseline: 236.2501x; 1.2134x over previous
import jax
import jax.numpy as jnp
from jax.experimental import pallas as pl
from jax.experimental.pallas import tpu as pltpu

NUM_CLASSES = 80
CPC = NUM_CLASSES // 2
CC = 8
IOU_THRESHOLD = 0.5
EPS = 1e-6
DB = 64
BIG = 1.0e9


def _map_kernel(starts_ref, det_ref, dl_ref, gt_ref, out_ref, vb_ref, tprow_ref):
    core = pl.program_id(0)
    cbase = core * CPC

    g = gt_ref[...]
    gx1 = g[0:1, :]
    gy1 = g[1:2, :]
    gx2 = g[2:3, :]
    gy2 = g[3:4, :]
    glab = g[4:5, :]
    gimg = g[5:6, :]
    garea = jnp.abs((gx2 - gx1) * (gy2 - gy1))
    G = g.shape[1]
    D = det_ref.shape[0]

    lane_g = jax.lax.broadcasted_iota(jnp.int32, (1, G), 1)

    s_lo = starts_ref[cbase]
    s_hi = jnp.where(cbase + CPC >= NUM_CLASSES, D,
                     starts_ref[jnp.minimum(cbase + CPC, NUM_CLASSES - 1)])
    b_lo = s_lo // DB
    b_hi = (s_hi + DB - 1) // DB

    def blockA(b, first_p):
        row0 = b * DB
        d = det_ref[pl.ds(row0, DB), :]
        dx1 = d[:, 0:1]
        dy1 = d[:, 1:2]
        dx2 = d[:, 2:3]
        dy2 = d[:, 3:4]
        dimg = d[:, 4:5]
        dlab = d[:, 5:6]
        darea = jnp.abs((dx2 - dx1) * (dy2 - dy1))

        ix1 = jnp.maximum(dx1, gx1)
        iy1 = jnp.maximum(dy1, gy1)
        ix2 = jnp.minimum(dx2, gx2)
        iy2 = jnp.minimum(dy2, gy2)
        inter = jnp.clip(ix2 - ix1, 0.0) * jnp.clip(iy2 - iy1, 0.0)
        iou = inter / (darea + garea - inter + EPS)
        valid = (glab == dlab) & (gimg == dimg)
        iou_m = jnp.where(valid, iou, -1.0)
        val = jnp.max(iou_m, axis=1, keepdims=True)
        best = jnp.argmax(iou_m, axis=1, keepdims=True)
        vb_ref[pl.ds(row0, DB), 0:1] = val
        vb_ref[pl.ds(row0, DB), 1:2] = best.astype(jnp.float32)

        p_col = (row0 + jax.lax.broadcasted_iota(jnp.int32, (DB, 1), 0)).astype(
            jnp.float32
        )
        claim = (lane_g == best) & (val > IOU_THRESHOLD)
        cand = jnp.where(claim, p_col, BIG)
        return jnp.minimum(first_p, jnp.min(cand, axis=0, keepdims=True))

    first_p = jax.lax.fori_loop(
        b_lo, b_hi, blockA, jnp.full((1, G), BIG, jnp.float32)
    )

    DT = 128
    eye = (
        jax.lax.broadcasted_iota(jnp.int32, (DT, DT), 0)
        == jax.lax.broadcasted_iota(jnp.int32, (DT, DT), 1)
    ).astype(jnp.float32)

    def blockT(b, _):
        row0 = pl.multiple_of(b * DT, DT)
        val = vb_ref[pl.ds(row0, DT), 0:1]
        best = vb_ref[pl.ds(row0, DT), 1:2].astype(jnp.int32)
        B = lane_g == best
        fpb = jnp.max(jnp.where(B, first_p, -1.0), axis=1, keepdims=True)
        p_col = (row0 + jax.lax.broadcasted_iota(jnp.int32, (DT, 1), 0)).astype(
            jnp.float32
        )
        tp = jnp.where((val > IOU_THRESHOLD) & (fpb == p_col), 1.0, 0.0)
        tp_row = jax.lax.dot_general(
            tp, eye, (((0,), (0,)), ((), ())),
            preferred_element_type=jnp.float32,
        )
        tprow_ref[0:1, pl.ds(row0, DT)] = tp_row
        return 0

    jax.lax.fori_loop(s_lo // DT, (s_hi + DT - 1) // DT, blockT, 0)

    dl = dl_ref[...]
    tp_row = tprow_ref[...]
    lane_d = jax.lax.broadcasted_iota(jnp.int32, (1, D), 1)

    ap_sum = jnp.float32(0.0)
    hg_cnt = jnp.float32(0.0)
    for cc in range(CPC // CC):
        cvec = cbase + cc * CC + jax.lax.broadcasted_iota(jnp.int32, (CC, 1), 0)
        onehot = dl == cvec
        tpm = jnp.where(onehot, tp_row, 0.0)
        fpm = jnp.where(onehot, 1.0 - tp_row, 0.0)
        sh = 1
        while sh < D:
            z = jnp.zeros((CC, sh), jnp.float32)
            tpm = tpm + jnp.concatenate([z, tpm[:, : D - sh]], axis=1)
            fpm = fpm + jnp.concatenate([z, fpm[:, : D - sh]], axis=1)
            sh *= 2
        n_gt = jnp.sum(
            jnp.where(glab == cvec.astype(jnp.float32), 1.0, 0.0),
            axis=1,
            keepdims=True,
        )
        recall = tpm / (n_gt + EPS)
        denom = tpm + fpm
        prec = jnp.where(denom > 0, tpm / (denom + EPS), 1.0)
        z1 = jnp.zeros((CC, 1), jnp.float32)
        r_prev = jnp.concatenate([z1, recall[:, : D - 1]], axis=1)
        p_prev = jnp.concatenate([z1 + 1.0, prec[:, : D - 1]], axis=1)
        ap = jnp.sum((recall - r_prev) * (prec + p_prev) * 0.5, axis=1, keepdims=True)
        has_gt = n_gt > 0
        ap_sum = ap_sum + jnp.sum(jnp.where(has_gt, ap, 0.0))
        hg_cnt = hg_cnt + jnp.sum(jnp.where(has_gt, 1.0, 0.0))

    lane = jax.lax.broadcasted_iota(jnp.int32, (1, 8, 128), 2)
    out_ref[...] = jnp.where(lane == 0, ap_sum, jnp.where(lane == 1, hg_cnt, 0.0))


def kernel(det_boxes, det_scores, det_labels, det_image_ids, gt_boxes, gt_labels, gt_image_ids):
    D = det_boxes.shape[0]
    G = gt_boxes.shape[0]

    order = jnp.lexsort((-det_scores, det_labels))
    db = det_boxes[order]
    dl = det_labels[order].astype(jnp.int32)
    di = det_image_ids[order]

    det_pack = jnp.concatenate(
        [
            db,
            di.astype(jnp.float32)[:, None],
            dl.astype(jnp.float32)[:, None],
            jnp.zeros((D, 2), jnp.float32),
        ],
        axis=1,
    )
    gt_pack = jnp.concatenate(
        [
            gt_boxes.T,
            gt_labels.astype(jnp.float32)[None, :],
            gt_image_ids.astype(jnp.float32)[None, :],
            jnp.zeros((2, G), jnp.float32),
        ],
        axis=0,
    )

    classes = jnp.arange(NUM_CLASSES, dtype=jnp.int32)
    starts = jnp.searchsorted(dl, classes).astype(jnp.int32)

    out = pl.pallas_call(
        _map_kernel,
        grid=(2,),
        in_specs=[
            pl.BlockSpec(memory_space=pltpu.SMEM),
            pl.BlockSpec((D, 8), lambda i: (0, 0)),
            pl.BlockSpec((1, D), lambda i: (0, 0)),
            pl.BlockSpec((8, G), lambda i: (0, 0)),
        ],
        out_specs=pl.BlockSpec((1, 8, 128), lambda i: (i, 0, 0)),
        out_shape=jax.ShapeDtypeStruct((2, 8, 128), jnp.float32),
        scratch_shapes=[
            pltpu.VMEM((D, 8), jnp.float32),
            pltpu.VMEM((1, D), jnp.float32),
        ],
        compiler_params=pltpu.CompilerParams(
            dimension_semantics=("parallel",),
        ),
    )(starts, det_pack, dl[None, :], gt_pack)

    ap_total = out[0, 0, 0] + out[1, 0, 0]
    cnt_total = out[0, 0, 1] + out[1, 0, 1]
    return ap_total / jnp.maximum(cnt_total, 1.0)


# speedup vs baseline: 320.9612x; 1.3586x over previous
import jax
import jax.numpy as jnp
from jax.experimental import pallas as pl
from jax.experimental.pallas import tpu as pltpu

NUM_CLASSES = 80
CPC = NUM_CLASSES // 2
CC = 8
IOU_THRESHOLD = 0.5
EPS = 1e-6
DB = 64
DT = 128
W = 512
BIG = 1.0e9


def _map_kernel(starts_ref, gstarts_ref, dls_ref, det_ref, dl_ref, gt_ref,
                out_ref, vb_ref, tprow_ref, fp_ref):
    core = pl.program_id(0)
    cbase = core * CPC

    g = gt_ref[...]
    glab = g[4:5, :]
    G = g.shape[1]
    D = det_ref.shape[0]

    s_lo = starts_ref[cbase]
    s_hi = jnp.where(cbase + CPC >= NUM_CLASSES, D,
                     starts_ref[jnp.minimum(cbase + CPC, NUM_CLASSES - 1)])

    fp_ref[...] = jnp.full((1, G), BIG, jnp.float32)

    def window(row0, rows):
        cls_lo = dls_ref[row0]
        cls_hi = dls_ref[row0 + rows - 1]
        wlo = (gstarts_ref[cls_lo] >> 7) << 7
        whi = gstarts_ref[cls_hi + 1]
        nch = jnp.maximum((whi - wlo + W - 1) // W, 1)
        return wlo, nch

    def blockA(b, _):
        row0 = b * DB
        d = det_ref[pl.ds(row0, DB), :]
        dx1 = d[:, 0:1]
        dy1 = d[:, 1:2]
        dx2 = d[:, 2:3]
        dy2 = d[:, 3:4]
        dimg = d[:, 4:5]
        dlab = d[:, 5:6]
        darea = jnp.abs((dx2 - dx1) * (dy2 - dy1))
        wlo, nch = window(row0, DB)

        def chunk(c, carry):
            m, bg = carry
            col0 = pl.multiple_of(jnp.minimum(wlo + c * W, G - W), 128)
            gc = gt_ref[:, pl.ds(col0, W)]
            ix1 = jnp.maximum(dx1, gc[0:1, :])
            iy1 = jnp.maximum(dy1, gc[1:2, :])
            ix2 = jnp.minimum(dx2, gc[2:3, :])
            iy2 = jnp.minimum(dy2, gc[3:4, :])
            inter = jnp.clip(ix2 - ix1, 0.0) * jnp.clip(iy2 - iy1, 0.0)
            garea = jnp.abs((gc[2:3, :] - gc[0:1, :]) * (gc[3:4, :] - gc[1:2, :]))
            iou = inter / (darea + garea - inter + EPS)
            valid = (gc[4:5, :] == dlab) & (gc[5:6, :] == dimg)
            iou_m = jnp.where(valid, iou, -1.0)
            lm = jnp.max(iou_m, axis=1, keepdims=True)
            lb = jnp.argmax(iou_m, axis=1, keepdims=True) + col0
            upd = lm > m
            return jnp.where(upd, lm, m), jnp.where(upd, lb, bg)

        m, bg = jax.lax.fori_loop(
            0, nch, chunk,
            (jnp.full((DB, 1), -1.0, jnp.float32), jnp.zeros((DB, 1), jnp.int32)),
        )
        vb_ref[pl.ds(row0, DB), 0:1] = m
        vb_ref[pl.ds(row0, DB), 1:2] = bg.astype(jnp.float32)

        p_col = (row0 + jax.lax.broadcasted_iota(jnp.int32, (DB, 1), 0)).astype(
            jnp.float32
        )
        claim_row = m > IOU_THRESHOLD

        def chunk2(c, _):
            col0 = pl.multiple_of(jnp.minimum(wlo + c * W, G - W), 128)
            lane_c = col0 + jax.lax.broadcasted_iota(jnp.int32, (1, W), 1)
            claim = (lane_c == bg) & claim_row
            cand = jnp.where(claim, p_col, BIG)
            fp_ref[0:1, pl.ds(col0, W)] = jnp.minimum(
                fp_ref[0:1, pl.ds(col0, W)],
                jnp.min(cand, axis=0, keepdims=True),
            )
            return 0

        jax.lax.fori_loop(0, nch, chunk2, 0)
        return 0

    jax.lax.fori_loop(s_lo // DB, (s_hi + DB - 1) // DB, blockA, 0)

    eye = (
        jax.lax.broadcasted_iota(jnp.int32, (DT, DT), 0)
        == jax.lax.broadcasted_iota(jnp.int32, (DT, DT), 1)
    ).astype(jnp.float32)

    def blockT(b, _):
        row0 = pl.multiple_of(b * DT, DT)
        val = vb_ref[pl.ds(row0, DT), 0:1]
        best = vb_ref[pl.ds(row0, DT), 1:2].astype(jnp.int32)
        wlo, nch = window(row0, DT)

        def chunk(c, fpb):
            col0 = pl.multiple_of(jnp.minimum(wlo + c * W, G - W), 128)
            lane_c = col0 + jax.lax.broadcasted_iota(jnp.int32, (1, W), 1)
            fpw = fp_ref[0:1, pl.ds(col0, W)]
            hit = jnp.max(
                jnp.where(lane_c == best, fpw, -1.0), axis=1, keepdims=True
            )
            return jnp.maximum(fpb, hit)

        fpb = jax.lax.fori_loop(
            0, nch, chunk, jnp.full((DT, 1), -1.0, jnp.float32)
        )
        p_col = (row0 + jax.lax.broadcasted_iota(jnp.int32, (DT, 1), 0)).astype(
            jnp.float32
        )
        tp = jnp.where((val > IOU_THRESHOLD) & (fpb == p_col), 1.0, 0.0)
        tp_row = jax.lax.dot_general(
            tp, eye, (((0,), (0,)), ((), ())),
            preferred_element_type=jnp.float32,
        )
        tprow_ref[0:1, pl.ds(row0, DT)] = tp_row
        return 0

    jax.lax.fori_loop(s_lo // DT, (s_hi + DT - 1) // DT, blockT, 0)

    dl = dl_ref[...]
    tp_row = tprow_ref[...]

    ap_sum = jnp.float32(0.0)
    hg_cnt = jnp.float32(0.0)
    for cc in range(CPC // CC):
        cvec = cbase + cc * CC + jax.lax.broadcasted_iota(jnp.int32, (CC, 1), 0)
        onehot = dl == cvec
        tpm = jnp.where(onehot, tp_row, 0.0)
        fpm = jnp.where(onehot, 1.0 - tp_row, 0.0)
        sh = 1
        while sh < D:
            z = jnp.zeros((CC, sh), jnp.float32)
            tpm = tpm + jnp.concatenate([z, tpm[:, : D - sh]], axis=1)
            fpm = fpm + jnp.concatenate([z, fpm[:, : D - sh]], axis=1)
            sh *= 2
        n_gt = jnp.sum(
            jnp.where(glab == cvec.astype(jnp.float32), 1.0, 0.0),
            axis=1,
            keepdims=True,
        )
        recall = tpm / (n_gt + EPS)
        denom = tpm + fpm
        prec = jnp.where(denom > 0, tpm / (denom + EPS), 1.0)
        z1 = jnp.zeros((CC, 1), jnp.float32)
        r_prev = jnp.concatenate([z1, recall[:, : D - 1]], axis=1)
        p_prev = jnp.concatenate([z1 + 1.0, prec[:, : D - 1]], axis=1)
        ap = jnp.sum((recall - r_prev) * (prec + p_prev) * 0.5, axis=1, keepdims=True)
        has_gt = n_gt > 0
        ap_sum = ap_sum + jnp.sum(jnp.where(has_gt, ap, 0.0))
        hg_cnt = hg_cnt + jnp.sum(jnp.where(has_gt, 1.0, 0.0))

    lane = jax.lax.broadcasted_iota(jnp.int32, (1, 8, 128), 2)
    out_ref[...] = jnp.where(lane == 0, ap_sum, jnp.where(lane == 1, hg_cnt, 0.0))


def kernel(det_boxes, det_scores, det_labels, det_image_ids, gt_boxes, gt_labels, gt_image_ids):
    D = det_boxes.shape[0]
    G = gt_boxes.shape[0]

    order = jnp.lexsort((-det_scores, det_labels))
    db = det_boxes[order]
    dl = det_labels[order].astype(jnp.int32)
    di = det_image_ids[order]

    gorder = jnp.argsort(gt_labels, stable=True)
    gb = gt_boxes[gorder]
    gl = gt_labels[gorder].astype(jnp.int32)
    gi = gt_image_ids[gorder]

    det_pack = jnp.concatenate(
        [
            db,
            di.astype(jnp.float32)[:, None],
            dl.astype(jnp.float32)[:, None],
            jnp.zeros((D, 2), jnp.float32),
        ],
        axis=1,
    )
    gt_pack = jnp.concatenate(
        [
            gb.T,
            gl.astype(jnp.float32)[None, :],
            gi.astype(jnp.float32)[None, :],
            jnp.zeros((2, G), jnp.float32),
        ],
        axis=0,
    )

    classes = jnp.arange(NUM_CLASSES, dtype=jnp.int32)
    starts = jnp.searchsorted(dl, classes).astype(jnp.int32)
    gstarts = jnp.searchsorted(gl, jnp.arange(NUM_CLASSES + 1, dtype=jnp.int32)).astype(
        jnp.int32
    )

    out = pl.pallas_call(
        _map_kernel,
        grid=(2,),
        in_specs=[
            pl.BlockSpec(memory_space=pltpu.SMEM),
            pl.BlockSpec(memory_space=pltpu.SMEM),
            pl.BlockSpec(memory_space=pltpu.SMEM),
            pl.BlockSpec((D, 8), lambda i: (0, 0)),
            pl.BlockSpec((1, D), lambda i: (0, 0)),
            pl.BlockSpec((8, G), lambda i: (0, 0)),
        ],
        out_specs=pl.BlockSpec((1, 8, 128), lambda i: (i, 0, 0)),
        out_shape=jax.ShapeDtypeStruct((2, 8, 128), jnp.float32),
        scratch_shapes=[
            pltpu.VMEM((D, 8), jnp.float32),
            pltpu.VMEM((1, D), jnp.float32),
            pltpu.VMEM((1, G), jnp.float32),
        ],
        compiler_params=pltpu.CompilerParams(
            dimension_semantics=("parallel",),
        ),
    )(starts, gstarts, dl, det_pack, dl[None, :], gt_pack)

    ap_total = out[0, 0, 0] + out[1, 0, 0]
    cnt_total = out[0, 0, 1] + out[1, 0, 1]
    return ap_total / jnp.maximum(cnt_total, 1.0)


# speedup vs baseline: 325.6347x; 1.0146x over previous
import jax
import jax.numpy as jnp
from jax.experimental import pallas as pl
from jax.experimental.pallas import tpu as pltpu

NUM_CLASSES = 80
CPC = NUM_CLASSES // 2
CC = 8
IOU_THRESHOLD = 0.5
EPS = 1e-6
DB = 128
DT = 256
W = 512
BIG = 1.0e9


def _map_kernel(starts_ref, gstarts_ref, dls_ref, det_ref, dl_ref, gt_ref,
                out_ref, vb_ref, tprow_ref, fp_ref):
    core = pl.program_id(0)
    cbase = core * CPC

    g = gt_ref[...]
    glab = g[4:5, :]
    G = g.shape[1]
    D = det_ref.shape[0]

    s_lo = starts_ref[cbase]
    s_hi = jnp.where(cbase + CPC >= NUM_CLASSES, D,
                     starts_ref[jnp.minimum(cbase + CPC, NUM_CLASSES - 1)])

    fp_ref[...] = jnp.full((1, G), BIG, jnp.float32)

    def window(row0, rows):
        cls_lo = dls_ref[row0]
        cls_hi = dls_ref[row0 + rows - 1]
        wlo = (gstarts_ref[cls_lo] >> 7) << 7
        whi = gstarts_ref[cls_hi + 1]
        nch = jnp.maximum((whi - wlo + W - 1) // W, 1)
        return wlo, nch

    def blockA(b, _):
        row0 = b * DB
        d = det_ref[pl.ds(row0, DB), :]
        dx1 = d[:, 0:1]
        dy1 = d[:, 1:2]
        dx2 = d[:, 2:3]
        dy2 = d[:, 3:4]
        dimg = d[:, 4:5]
        dlab = d[:, 5:6]
        darea = jnp.abs((dx2 - dx1) * (dy2 - dy1))
        wlo, nch = window(row0, DB)

        def chunk(c, carry):
            m, bg = carry
            col0 = pl.multiple_of(jnp.minimum(wlo + c * W, G - W), 128)
            gc = gt_ref[:, pl.ds(col0, W)]
            ix1 = jnp.maximum(dx1, gc[0:1, :])
            iy1 = jnp.maximum(dy1, gc[1:2, :])
            ix2 = jnp.minimum(dx2, gc[2:3, :])
            iy2 = jnp.minimum(dy2, gc[3:4, :])
            inter = jnp.clip(ix2 - ix1, 0.0) * jnp.clip(iy2 - iy1, 0.0)
            garea = jnp.abs((gc[2:3, :] - gc[0:1, :]) * (gc[3:4, :] - gc[1:2, :]))
            iou = inter / (darea + garea - inter + EPS)
            valid = (gc[4:5, :] == dlab) & (gc[5:6, :] == dimg)
            iou_m = jnp.where(valid, iou, -1.0)
            lm = jnp.max(iou_m, axis=1, keepdims=True)
            lb = jnp.argmax(iou_m, axis=1, keepdims=True) + col0
            upd = lm > m
            return jnp.where(upd, lm, m), jnp.where(upd, lb, bg)

        m, bg = jax.lax.fori_loop(
            0, nch, chunk,
            (jnp.full((DB, 1), -1.0, jnp.float32), jnp.zeros((DB, 1), jnp.int32)),
        )
        vb_ref[pl.ds(row0, DB), 0:1] = m
        vb_ref[pl.ds(row0, DB), 1:2] = bg.astype(jnp.float32)

        p_col = (row0 + jax.lax.broadcasted_iota(jnp.int32, (DB, 1), 0)).astype(
            jnp.float32
        )
        claim_row = m > IOU_THRESHOLD

        def chunk2(c, _):
            col0 = pl.multiple_of(jnp.minimum(wlo + c * W, G - W), 128)
            lane_c = col0 + jax.lax.broadcasted_iota(jnp.int32, (1, W), 1)
            claim = (lane_c == bg) & claim_row
            cand = jnp.where(claim, p_col, BIG)
            fp_ref[0:1, pl.ds(col0, W)] = jnp.minimum(
                fp_ref[0:1, pl.ds(col0, W)],
                jnp.min(cand, axis=0, keepdims=True),
            )
            return 0

        jax.lax.fori_loop(0, nch, chunk2, 0)
        return 0

    jax.lax.fori_loop(s_lo // DB, (s_hi + DB - 1) // DB, blockA, 0)

    eye = (
        jax.lax.broadcasted_iota(jnp.int32, (DT, DT), 0)
        == jax.lax.broadcasted_iota(jnp.int32, (DT, DT), 1)
    ).astype(jnp.float32)

    def blockT(b, _):
        row0 = pl.multiple_of(b * DT, DT)
        val = vb_ref[pl.ds(row0, DT), 0:1]
        best = vb_ref[pl.ds(row0, DT), 1:2].astype(jnp.int32)
        wlo, nch = window(row0, DT)

        def chunk(c, fpb):
            col0 = pl.multiple_of(jnp.minimum(wlo + c * W, G - W), 128)
            lane_c = col0 + jax.lax.broadcasted_iota(jnp.int32, (1, W), 1)
            fpw = fp_ref[0:1, pl.ds(col0, W)]
            hit = jnp.max(
                jnp.where(lane_c == best, fpw, -1.0), axis=1, keepdims=True
            )
            return jnp.maximum(fpb, hit)

        fpb = jax.lax.fori_loop(
            0, nch, chunk, jnp.full((DT, 1), -1.0, jnp.float32)
        )
        p_col = (row0 + jax.lax.broadcasted_iota(jnp.int32, (DT, 1), 0)).astype(
            jnp.float32
        )
        tp = jnp.where((val > IOU_THRESHOLD) & (fpb == p_col), 1.0, 0.0)
        tp_row = jax.lax.dot_general(
            tp, eye, (((0,), (0,)), ((), ())),
            preferred_element_type=jnp.float32,
        )
        tprow_ref[0:1, pl.ds(row0, DT)] = tp_row
        return 0

    jax.lax.fori_loop(s_lo // DT, (s_hi + DT - 1) // DT, blockT, 0)

    dl = dl_ref[...]
    tp_row = tprow_ref[...]

    ap_sum = jnp.float32(0.0)
    hg_cnt = jnp.float32(0.0)
    for cc in range(CPC // CC):
        cvec = cbase + cc * CC + jax.lax.broadcasted_iota(jnp.int32, (CC, 1), 0)
        onehot = dl == cvec
        tpm = jnp.where(onehot, tp_row, 0.0)
        fpm = jnp.where(onehot, 1.0 - tp_row, 0.0)
        sh = 1
        while sh < D:
            z = jnp.zeros((CC, sh), jnp.float32)
            tpm = tpm + jnp.concatenate([z, tpm[:, : D - sh]], axis=1)
            fpm = fpm + jnp.concatenate([z, fpm[:, : D - sh]], axis=1)
            sh *= 2
        n_gt = jnp.sum(
            jnp.where(glab == cvec.astype(jnp.float32), 1.0, 0.0),
            axis=1,
            keepdims=True,
        )
        recall = tpm / (n_gt + EPS)
        denom = tpm + fpm
        prec = jnp.where(denom > 0, tpm / (denom + EPS), 1.0)
        z1 = jnp.zeros((CC, 1), jnp.float32)
        r_prev = jnp.concatenate([z1, recall[:, : D - 1]], axis=1)
        p_prev = jnp.concatenate([z1 + 1.0, prec[:, : D - 1]], axis=1)
        ap = jnp.sum((recall - r_prev) * (prec + p_prev) * 0.5, axis=1, keepdims=True)
        has_gt = n_gt > 0
        ap_sum = ap_sum + jnp.sum(jnp.where(has_gt, ap, 0.0))
        hg_cnt = hg_cnt + jnp.sum(jnp.where(has_gt, 1.0, 0.0))

    lane = jax.lax.broadcasted_iota(jnp.int32, (1, 8, 128), 2)
    out_ref[...] = jnp.where(lane == 0, ap_sum, jnp.where(lane == 1, hg_cnt, 0.0))


def kernel(det_boxes, det_scores, det_labels, det_image_ids, gt_boxes, gt_labels, gt_image_ids):
    D = det_boxes.shape[0]
    G = gt_boxes.shape[0]

    order = jnp.lexsort((-det_scores, det_labels))
    det_all = jnp.concatenate(
        [
            det_boxes,
            det_image_ids.astype(jnp.float32)[:, None],
            det_labels.astype(jnp.float32)[:, None],
            jnp.zeros((D, 2), jnp.float32),
        ],
        axis=1,
    )
    det_pack = det_all[order]
    dl = det_labels[order].astype(jnp.int32)

    gorder = jnp.argsort(gt_labels, stable=True)
    gt_all = jnp.concatenate(
        [
            gt_boxes,
            gt_labels.astype(jnp.float32)[:, None],
            gt_image_ids.astype(jnp.float32)[:, None],
            jnp.zeros((G, 2), jnp.float32),
        ],
        axis=1,
    )
    gt_pack = gt_all[gorder].T
    gl = gt_labels[gorder].astype(jnp.int32)

    classes = jnp.arange(NUM_CLASSES, dtype=jnp.int32)
    starts = jnp.searchsorted(dl, classes).astype(jnp.int32)
    gstarts = jnp.searchsorted(gl, jnp.arange(NUM_CLASSES + 1, dtype=jnp.int32)).astype(
        jnp.int32
    )

    out = pl.pallas_call(
        _map_kernel,
        grid=(2,),
        in_specs=[
            pl.BlockSpec(memory_space=pltpu.SMEM),
            pl.BlockSpec(memory_space=pltpu.SMEM),
            pl.BlockSpec(memory_space=pltpu.SMEM),
            pl.BlockSpec((D, 8), lambda i: (0, 0)),
            pl.BlockSpec((1, D), lambda i: (0, 0)),
            pl.BlockSpec((8, G), lambda i: (0, 0)),
        ],
        out_specs=pl.BlockSpec((1, 8, 128), lambda i: (i, 0, 0)),
        out_shape=jax.ShapeDtypeStruct((2, 8, 128), jnp.float32),
        scratch_shapes=[
            pltpu.VMEM((D, 8), jnp.float32),
            pltpu.VMEM((1, D), jnp.float32),
            pltpu.VMEM((1, G), jnp.float32),
        ],
        compiler_params=pltpu.CompilerParams(
            dimension_semantics=("parallel",),
        ),
    )(starts, gstarts, dl, det_pack, dl[None, :], gt_pack)

    ap_total = out[0, 0, 0] + out[1, 0, 0]
    cnt_total = out[0, 0, 1] + out[1, 0, 1]
    return ap_total / jnp.maximum(cnt_total, 1.0)
